# Initial kernel scaffold; baseline (speedup 1.0000x reference)
import functools

import jax
import jax.numpy as jnp
from jax import lax
from jax.experimental import pallas as pl
from jax.experimental.pallas import tpu as pltpu
from jax.experimental.pallas import tpu_sc as plsc

N = 10000
E = 320000
D = 128
HD = 128
L = 2

NC = 2
NS = 16
NW = NC * NS
EPW = E // NW
CH = 80
NCHUNK = EPW // CH
RPT = N // NS

_mesh = plsc.VectorSubcoreMesh(core_axis_name="c", subcore_axis_name="s")


@functools.partial(
    pl.kernel,
    out_type=jax.ShapeDtypeStruct((NC, N, 8), jnp.float32),
    mesh=_mesh,
    scratch_types=[
        pltpu.VMEM((CH,), jnp.int32),
        pltpu.VMEM((CH, 8), jnp.float32),
        pltpu.VMEM_SHARED((N, 8), jnp.float32),
    ],
)
def _deg_kernel(dst_hbm, ones_hbm, zeros_hbm, out_hbm, dst_v, ones_v, acc):
    cid = lax.axis_index("c")
    sid = lax.axis_index("s")
    wid = cid * NS + sid
    r0 = sid * RPT
    pltpu.sync_copy(zeros_hbm.at[pl.ds(r0, RPT)], acc.at[pl.ds(r0, RPT)])
    pltpu.sync_copy(ones_hbm.at[pl.ds(0, CH)], ones_v)
    plsc.subcore_barrier()

    def body(j, carry):
        off = (wid * NCHUNK + j) * CH
        pltpu.sync_copy(dst_hbm.at[pl.ds(off, CH)], dst_v)
        pltpu.sync_copy(ones_v, acc.at[dst_v], add=True)
        return carry

    lax.fori_loop(0, NCHUNK, body, 0)
    plsc.subcore_barrier()
    pltpu.sync_copy(acc.at[pl.ds(r0, RPT)], out_hbm.at[cid, pl.ds(r0, RPT)])


@functools.partial(
    pl.kernel,
    out_type=jax.ShapeDtypeStruct((NC, N, HD), jnp.float32),
    mesh=_mesh,
    scratch_types=[
        pltpu.VMEM((CH,), jnp.int32),
        pltpu.VMEM((CH,), jnp.int32),
        pltpu.VMEM((CH, HD), jnp.float32),
        pltpu.VMEM_SHARED((N, HD), jnp.float32),
        pltpu.SemaphoreType.DMA,
    ],
)
def _prop_kernel(s_hbm, src_hbm, dst_hbm, zeros_hbm, out_hbm,
                 src_v, dst_v, rows_v, acc, sem):
    cid = lax.axis_index("c")
    sid = lax.axis_index("s")
    wid = cid * NS + sid
    r0 = sid * RPT

    @pl.when(cid == 0)
    def _():
        pltpu.sync_copy(s_hbm.at[pl.ds(r0, RPT)], acc.at[pl.ds(r0, RPT)])

    @pl.when(cid != 0)
    def _():
        pltpu.sync_copy(zeros_hbm.at[pl.ds(r0, RPT)], acc.at[pl.ds(r0, RPT)])

    plsc.subcore_barrier()

    def body(j, carry):
        off = (wid * NCHUNK + j) * CH
        pltpu.sync_copy(src_hbm.at[pl.ds(off, CH)], src_v)
        pltpu.sync_copy(dst_hbm.at[pl.ds(off, CH)], dst_v)
        pltpu.async_copy(s_hbm.at[src_v], rows_v, sem).wait()
        pltpu.sync_copy(rows_v, acc.at[dst_v], add=True)
        return carry

    lax.fori_loop(0, NCHUNK, body, 0)
    plsc.subcore_barrier()
    pltpu.sync_copy(acc.at[pl.ds(r0, RPT)], out_hbm.at[cid, pl.ds(r0, RPT)])


R = 1000
_GRID = N // R


def _dinv_of(degp):
    deg = degp[0, :, 0:1] + degp[1, :, 0:1] + 1.0
    return lax.rsqrt(deg)


def _stage_a_body(inp_ref, h_ref, degp_ref, wz_ref, wr_ref, sz_ref, sr_ref):
    dinv = _dinv_of(degp_ref[...])
    xi = inp_ref[...]
    hi = h_ref[...]
    cz = (jnp.dot(xi, wz_ref[0], preferred_element_type=jnp.float32)
          + jnp.dot(hi, wz_ref[1], preferred_element_type=jnp.float32))
    cr = (jnp.dot(xi, wr_ref[0], preferred_element_type=jnp.float32)
          + jnp.dot(hi, wr_ref[1], preferred_element_type=jnp.float32))
    sz_ref[...] = dinv * cz
    sr_ref[...] = dinv * cr


def _stage_b_body(gz_ref, gr_ref, degp_ref, inp_ref, h_ref, wh_ref,
                  bz_ref, br_ref, sh_ref, z_ref):
    dinv = _dinv_of(degp_ref[...])
    z = jax.nn.sigmoid(dinv * (gz_ref[0] + gz_ref[1]) + bz_ref[...])
    r = jax.nn.sigmoid(dinv * (gr_ref[0] + gr_ref[1]) + br_ref[...])
    ch = (jnp.dot(inp_ref[...], wh_ref[0], preferred_element_type=jnp.float32)
          + jnp.dot(r * h_ref[...], wh_ref[1], preferred_element_type=jnp.float32))
    sh_ref[...] = dinv * ch
    z_ref[...] = z


def _stage_c_body(gh_ref, degp_ref, z_ref, h_ref, bh_ref, out_ref):
    dinv = _dinv_of(degp_ref[...])
    h_tilde = jnp.tanh(dinv * (gh_ref[0] + gh_ref[1]) + bh_ref[...])
    z = z_ref[...]
    out_ref[...] = z * h_ref[...] + (1.0 - z) * h_tilde


_row = pl.BlockSpec((R, HD), lambda i: (i, 0))
_gp = pl.BlockSpec((NC, R, HD), lambda i: (0, i, 0))
_dg = pl.BlockSpec((NC, R, 8), lambda i: (0, i, 0))
_wt = pl.BlockSpec((NC, D, HD), lambda i: (0, 0, 0))
_bs = pl.BlockSpec((1, HD), lambda i: (0, 0))
_o2 = jax.ShapeDtypeStruct((N, HD), jnp.float32)

_stage_a = pl.pallas_call(
    _stage_a_body, grid=(_GRID,),
    in_specs=[_row, _row, _dg, _wt, _wt],
    out_specs=[_row, _row], out_shape=[_o2, _o2])

_stage_b = pl.pallas_call(
    _stage_b_body, grid=(_GRID,),
    in_specs=[_gp, _gp, _dg, _row, _row, _wt, _bs, _bs],
    out_specs=[_row, _row], out_shape=[_o2, _o2])

_stage_c = pl.pallas_call(
    _stage_c_body, grid=(_GRID,),
    in_specs=[_gp, _dg, _row, _row, _bs],
    out_specs=_row, out_shape=_o2)


def kernel(x, h, edge_index, Wxz, Whz, Wxr, Whr, Wxh, Whh,
           bxz, bhz, bxr, bhr, bxh, bhh):
    src = edge_index[0]
    dst = edge_index[1]
    zeros128 = jnp.zeros((N, HD), jnp.float32)
    zeros8 = jnp.zeros((N, 8), jnp.float32)
    ones_ch = jnp.ones((CH, 8), jnp.float32)

    degp = _deg_kernel(dst, ones_ch, zeros8)

    inp = x
    outs = []
    for i in range(L):
        wz = jnp.stack([Wxz[i], Whz[i]])
        wr = jnp.stack([Wxr[i], Whr[i]])
        wh = jnp.stack([Wxh[i], Whh[i]])
        bz = (bxz[i] + bhz[i])[None, :]
        br = (bxr[i] + bhr[i])[None, :]
        bh = (bxh[i] + bhh[i])[None, :]
        sz, sr = _stage_a(inp, h[i], degp, wz, wr)
        gz = _prop_kernel(sz, src, dst, zeros128)
        gr = _prop_kernel(sr, src, dst, zeros128)
        sh, z = _stage_b(gz, gr, degp, inp, h[i], wh, bz, br)
        gh = _prop_kernel(sh, src, dst, zeros128)
        out = _stage_c(gh, degp, z, h[i], bh)
        outs.append(out)
        inp = out
    return jnp.stack(outs, axis=0)


# speedup vs baseline: 13.8593x; 13.8593x over previous
import functools

import jax
import jax.numpy as jnp
from jax import lax
from jax.experimental import pallas as pl
from jax.experimental.pallas import tpu as pltpu
from jax.experimental.pallas import tpu_sc as plsc

N = 10000
E = 320000
D = 128
HD = 128
L = 2

NC = 2
NS = 16
NW = NC * NS
EPW = E // NW
CH = 80
NCHUNK = EPW // CH
NP = 10240
RPT = NP // NS

_mesh = plsc.VectorSubcoreMesh(core_axis_name="c", subcore_axis_name="s")


@functools.partial(
    pl.kernel,
    out_type=jax.ShapeDtypeStruct((NC, NP, 8), jnp.float32),
    mesh=_mesh,
    scratch_types=[
        pltpu.VMEM((CH,), jnp.int32),
        pltpu.VMEM((CH, 8), jnp.float32),
        pltpu.VMEM_SHARED((NP, 8), jnp.float32),
    ],
)
def _deg_kernel(dst_hbm, ones_hbm, zeros_hbm, out_hbm, dst_v, ones_v, acc):
    cid = lax.axis_index("c")
    sid = lax.axis_index("s")
    wid = cid * NS + sid
    r0 = sid * RPT
    pltpu.sync_copy(zeros_hbm.at[pl.ds(r0, RPT)], acc.at[pl.ds(r0, RPT)])
    pltpu.sync_copy(ones_hbm.at[pl.ds(0, CH)], ones_v)
    plsc.subcore_barrier()

    def body(j, carry):
        off = (wid * NCHUNK + j) * CH
        pltpu.sync_copy(dst_hbm.at[pl.ds(off, CH)], dst_v)
        pltpu.sync_copy(ones_v, acc.at[dst_v], add=True)
        return carry

    lax.fori_loop(0, NCHUNK, body, 0)
    plsc.subcore_barrier()
    pltpu.sync_copy(acc.at[pl.ds(r0, RPT)], out_hbm.at[cid, pl.ds(r0, RPT)])


@functools.partial(
    pl.kernel,
    out_type=jax.ShapeDtypeStruct((NC, NP, HD), jnp.float32),
    mesh=_mesh,
    scratch_types=[
        pltpu.VMEM((CH,), jnp.int32),
        pltpu.VMEM((CH,), jnp.int32),
        pltpu.VMEM((CH, HD), jnp.float32),
        pltpu.VMEM_SHARED((NP, HD), jnp.float32),
        pltpu.SemaphoreType.DMA,
    ],
)
def _prop_kernel(s_hbm, src_hbm, dst_hbm, zeros_hbm, out_hbm,
                 src_v, dst_v, rows_v, acc, sem):
    cid = lax.axis_index("c")
    sid = lax.axis_index("s")
    wid = cid * NS + sid
    r0 = sid * RPT

    @pl.when(cid == 0)
    def _():
        pltpu.sync_copy(s_hbm.at[pl.ds(r0, RPT)], acc.at[pl.ds(r0, RPT)])

    @pl.when(cid != 0)
    def _():
        pltpu.sync_copy(zeros_hbm.at[pl.ds(r0, RPT)], acc.at[pl.ds(r0, RPT)])

    plsc.subcore_barrier()

    def body(j, carry):
        off = (wid * NCHUNK + j) * CH
        pltpu.sync_copy(src_hbm.at[pl.ds(off, CH)], src_v)
        pltpu.sync_copy(dst_hbm.at[pl.ds(off, CH)], dst_v)
        pltpu.async_copy(s_hbm.at[src_v], rows_v, sem).wait()
        pltpu.sync_copy(rows_v, acc.at[dst_v], add=True)
        return carry

    lax.fori_loop(0, NCHUNK, body, 0)
    plsc.subcore_barrier()
    pltpu.sync_copy(acc.at[pl.ds(r0, RPT)], out_hbm.at[cid, pl.ds(r0, RPT)])


R = 1000
_GRID = N // R


def _dinv_of(degp):
    deg = degp[0, :, 0:1] + degp[1, :, 0:1] + 1.0
    return lax.rsqrt(deg)


def _stage_a_body(inp_ref, h_ref, degp_ref, wz_ref, wr_ref, sz_ref, sr_ref):
    dinv = _dinv_of(degp_ref[...])
    xi = inp_ref[...]
    hi = h_ref[...]
    cz = (jnp.dot(xi, wz_ref[0], preferred_element_type=jnp.float32)
          + jnp.dot(hi, wz_ref[1], preferred_element_type=jnp.float32))
    cr = (jnp.dot(xi, wr_ref[0], preferred_element_type=jnp.float32)
          + jnp.dot(hi, wr_ref[1], preferred_element_type=jnp.float32))
    sz_ref[...] = dinv * cz
    sr_ref[...] = dinv * cr


def _stage_b_body(gz_ref, gr_ref, degp_ref, inp_ref, h_ref, wh_ref,
                  bz_ref, br_ref, sh_ref, z_ref):
    dinv = _dinv_of(degp_ref[...])
    z = jax.nn.sigmoid(dinv * (gz_ref[0] + gz_ref[1]) + bz_ref[...])
    r = jax.nn.sigmoid(dinv * (gr_ref[0] + gr_ref[1]) + br_ref[...])
    ch = (jnp.dot(inp_ref[...], wh_ref[0], preferred_element_type=jnp.float32)
          + jnp.dot(r * h_ref[...], wh_ref[1], preferred_element_type=jnp.float32))
    sh_ref[...] = dinv * ch
    z_ref[...] = z


def _stage_c_body(gh_ref, degp_ref, z_ref, h_ref, bh_ref, out_ref):
    dinv = _dinv_of(degp_ref[...])
    h_tilde = jnp.tanh(dinv * (gh_ref[0] + gh_ref[1]) + bh_ref[...])
    z = z_ref[...]
    out_ref[...] = z * h_ref[...] + (1.0 - z) * h_tilde


_row = pl.BlockSpec((R, HD), lambda i: (i, 0))
_gp = pl.BlockSpec((NC, R, HD), lambda i: (0, i, 0))
_dg = pl.BlockSpec((NC, R, 8), lambda i: (0, i, 0))
_wt = pl.BlockSpec((NC, D, HD), lambda i: (0, 0, 0))
_bs = pl.BlockSpec((1, HD), lambda i: (0, 0))
_o2 = jax.ShapeDtypeStruct((N, HD), jnp.float32)
_o2p = jax.ShapeDtypeStruct((NP, HD), jnp.float32)

_stage_a = pl.pallas_call(
    _stage_a_body, grid=(_GRID,),
    in_specs=[_row, _row, _dg, _wt, _wt],
    out_specs=[_row, _row], out_shape=[_o2p, _o2p])

_stage_b = pl.pallas_call(
    _stage_b_body, grid=(_GRID,),
    in_specs=[_gp, _gp, _dg, _row, _row, _wt, _bs, _bs],
    out_specs=[_row, _row], out_shape=[_o2p, _o2])

_stage_c = pl.pallas_call(
    _stage_c_body, grid=(_GRID,),
    in_specs=[_gp, _dg, _row, _row, _bs],
    out_specs=_row, out_shape=_o2)


def kernel(x, h, edge_index, Wxz, Whz, Wxr, Whr, Wxh, Whh,
           bxz, bhz, bxr, bhr, bxh, bhh):
    src = edge_index[0]
    dst = edge_index[1]
    zeros128 = jnp.zeros((NP, HD), jnp.float32)
    zeros8 = jnp.zeros((NP, 8), jnp.float32)
    ones_ch = jnp.ones((CH, 8), jnp.float32)

    degp = _deg_kernel(dst, ones_ch, zeros8)

    inp = x
    outs = []
    for i in range(L):
        wz = jnp.stack([Wxz[i], Whz[i]])
        wr = jnp.stack([Wxr[i], Whr[i]])
        wh = jnp.stack([Wxh[i], Whh[i]])
        bz = (bxz[i] + bhz[i])[None, :]
        br = (bxr[i] + bhr[i])[None, :]
        bh = (bxh[i] + bhh[i])[None, :]
        sz, sr = _stage_a(inp, h[i], degp, wz, wr)
        gz = _prop_kernel(sz, src, dst, zeros128)
        gr = _prop_kernel(sr, src, dst, zeros128)
        sh, z = _stage_b(gz, gr, degp, inp, h[i], wh, bz, br)
        gh = _prop_kernel(sh, src, dst, zeros128)
        out = _stage_c(gh, degp, z, h[i], bh)
        outs.append(out)
        inp = out
    return jnp.stack(outs, axis=0)
